# Initial kernel scaffold; baseline (speedup 1.0000x reference)
import jax
import jax.numpy as jnp
from jax.experimental import pallas as pl


def kernel(batch_of_context_vectors, embedding_weight):
    raise NotImplementedError("write your pallas kernel here")


# speedup vs baseline: 6.7183x; 6.7183x over previous
import functools

import jax
import jax.numpy as jnp
from jax import lax
from jax.experimental import pallas as pl
from jax.experimental.pallas import tpu as pltpu
from jax.experimental.pallas import tpu_sc as plsc

VOCAB = 1000
D = 64
B = 16384
C = 10
L = 16
NC, NS = 2, 16
NW = NC * NS
RW = B // NW
NB = 64
NI = RW // NB
IDX_PER_W = RW * C
G = 128

_LN2 = 0.6931471805599453


def _log_f32(x):
    bits = lax.bitcast_convert_type(x, jnp.int32)
    e = (bits >> 23) - 127
    mant = lax.bitcast_convert_type(
        (bits & jnp.int32(0x7FFFFF)) | jnp.int32(0x3F800000), jnp.float32)
    t = (mant - 1.0) / (mant + 1.0)
    t2 = t * t
    p = 1.0 + t2 * (0.3333333333 + t2 * (0.2 + t2 * 0.14285714285))
    return e.astype(jnp.float32) * _LN2 + 2.0 * t * p


def _sc_body(table_hbm, idx_hbm, out_hbm, idx_v, rows_v, sem):
    wid = lax.axis_index("s") * NC + lax.axis_index("c")
    base = wid * IDX_PER_W
    pltpu.sync_copy(idx_hbm.at[pl.ds(base, IDX_PER_W)], idx_v)

    for it in range(NI):
        copies = [
            pltpu.async_copy(
                table_hbm.at[idx_v.at[pl.ds(it * NB * C + g * G, G)]],
                rows_v.at[pl.ds(g * G, G)],
                sem,
            )
            for g in range(NB * C // G)
        ]
        for cp in copies:
            cp.wait()

        def row_body(b, _):
            g0 = b * C
            vs = []
            macc = jnp.full((L,), -jnp.inf, jnp.float32)
            for c in range(C):
                for k in range(D // L):
                    v = rows_v[g0 + c, pl.ds(k * L, L)]
                    vs.append(v)
                    macc = jnp.maximum(macc, v)
            m = jnp.broadcast_to(jnp.max(macc), (L,))
            zacc = jnp.zeros((L,), jnp.float32)
            for v in vs:
                zacc = zacc + jnp.exp(v - m)
            z = jnp.broadcast_to(jnp.sum(zacc), (L,))
            lse = m + _log_f32(z)
            i = 0
            for c in range(C):
                for k in range(D // L):
                    rows_v[g0 + c, pl.ds(k * L, L)] = vs[i] - lse
                    i += 1
            return _

        lax.fori_loop(0, NB, row_body, 0)
        pltpu.sync_copy(rows_v, out_hbm.at[pl.ds(base + it * NB * C, NB * C)])


@functools.partial(
    pl.kernel,
    out_type=jax.ShapeDtypeStruct((B * C, D), jnp.float32),
    mesh=plsc.VectorSubcoreMesh(core_axis_name="c", subcore_axis_name="s"),
    compiler_params=pltpu.CompilerParams(needs_layout_passes=False, use_tc_tiling_on_sc=False),
    scratch_types=[
        pltpu.VMEM((IDX_PER_W,), jnp.int32),
        pltpu.VMEM((NB * C, D), jnp.float32),
        pltpu.SemaphoreType.DMA,
    ],
)
def _sc_kernel(table_hbm, idx_hbm, out_hbm, idx_v, rows_v, sem):
    _sc_body(table_hbm, idx_hbm, out_hbm, idx_v, rows_v, sem)


def kernel(batch_of_context_vectors, embedding_weight):
    idx_flat = batch_of_context_vectors.reshape(-1)
    out = _sc_kernel(embedding_weight, idx_flat)
    return out.reshape(B, C * D)


# speedup vs baseline: 8.0730x; 1.2017x over previous
import functools

import jax
import jax.numpy as jnp
from jax import lax
from jax.experimental import pallas as pl
from jax.experimental.pallas import tpu as pltpu
from jax.experimental.pallas import tpu_sc as plsc

VOCAB = 1000
VPAD = 1024
D = 64
B = 16384
C = 10
L = 16
NC, NS = 2, 16
NW = NC * NS
RW = B // NW
NB = 32
NI = RW // NB
NBUF = 3
G = 80
NG = NB * C // G
EV = VPAD // NS

_LN2 = 0.6931471805599453


def _log_f32(x):
    bits = lax.bitcast_convert_type(x, jnp.int32)
    e = (bits >> 23) - 127
    mant = lax.bitcast_convert_type(
        (bits & jnp.int32(0x7FFFFF)) | jnp.int32(0x3F800000), jnp.float32)
    t = (mant - 1.0) / (mant + 1.0)
    t2 = t * t
    p = 1.0 + t2 * (0.3333333333 + t2 * (0.2 + t2 * 0.14285714285))
    return e.astype(jnp.float32) * _LN2 + 2.0 * t * p


def _sc_body(table_hbm, idx_hbm, idxt_hbm, out_hbm,
             idx_v, idxt_v, bufs, tstat_v, ms_loc, ms_v, lse_v,
             ms_sh, sem_g, sem_o):
    cid = lax.axis_index("c")
    sid = lax.axis_index("s")
    wid = sid * NC + cid
    base = wid * RW * C
    row0 = wid * RW

    pltpu.sync_copy(idx_hbm.at[pl.ds(base, RW * C)], idx_v)
    for c in range(C):
        pltpu.sync_copy(idxt_hbm.at[pl.ds(c * B + row0, RW)], idxt_v.at[c])

    def fire_gather(it, p):
        return [
            pltpu.async_copy(
                table_hbm.at[idx_v.at[pl.ds(it * NB * C + g * G, G)]],
                bufs[p].at[pl.ds(g * G, G)],
                sem_g[p],
            )
            for g in range(NG)
        ]

    g_pend = [fire_gather(p, p) for p in range(NBUF)]
    out_pend = [None] * NBUF

    pltpu.sync_copy(table_hbm.at[pl.ds(sid * EV, EV)], tstat_v)
    lanes = lax.broadcasted_iota(jnp.int32, (L,), 0)
    for grp in range(EV // L):
        rr = lanes + grp * L
        macc = plsc.load_gather(tstat_v, [rr, jnp.zeros((L,), jnp.int32)])
        for d in range(1, D):
            macc = jnp.maximum(
                macc, plsc.load_gather(tstat_v, [rr, jnp.full((L,), d, jnp.int32)]))
        sacc = jnp.zeros((L,), jnp.float32)
        for d in range(D):
            v = plsc.load_gather(tstat_v, [rr, jnp.full((L,), d, jnp.int32)])
            sacc = sacc + jnp.exp(v - macc)
        ms_loc[0, pl.ds(grp * L, L)] = macc
        ms_loc[1, pl.ds(grp * L, L)] = sacc
    pltpu.sync_copy(ms_loc.at[0], ms_sh.at[0, pl.ds(sid * EV, EV)])
    pltpu.sync_copy(ms_loc.at[1], ms_sh.at[1, pl.ds(sid * EV, EV)])
    plsc.subcore_barrier()
    pltpu.sync_copy(ms_sh, ms_v)

    for grp in range(RW // L):
        idxs = [idxt_v[c, pl.ds(grp * L, L)] for c in range(C)]
        ms = [plsc.load_gather(ms_v.at[0], [ix]) for ix in idxs]
        m16 = ms[0]
        for c in range(1, C):
            m16 = jnp.maximum(m16, ms[c])
        z16 = jnp.zeros((L,), jnp.float32)
        for c in range(C):
            s = plsc.load_gather(ms_v.at[1], [idxs[c]])
            z16 = z16 + s * jnp.exp(ms[c] - m16)
        lse_v[pl.ds(grp * L, L)] = 0.0 - (m16 + _log_f32(z16))

    for it in range(NI):
        p = it % NBUF

        for cp in g_pend[p]:
            cp.wait()

        def row_body(r, _):
            nls = plsc.load_gather(
                lse_v, [jnp.broadcast_to(it * NB + r, (L,)).astype(jnp.int32)])
            g0 = r * C
            for c in range(C):
                for k in range(D // L):
                    plsc.addupdate(bufs[p].at[g0 + c, pl.ds(k * L, L)], nls)
            return _

        lax.fori_loop(0, NB, row_body, 0)
        out_pend[p] = pltpu.async_copy(
            bufs[p], out_hbm.at[pl.ds(base + it * NB * C, NB * C)], sem_o[p])

        nxt = it + 2
        if NBUF <= nxt < NI:
            t = nxt % NBUF
            out_pend[t].wait()
            g_pend[t] = fire_gather(nxt, t)

    out_pend[(NI - 3) % NBUF].wait()
    out_pend[(NI - 2) % NBUF].wait()
    out_pend[(NI - 1) % NBUF].wait()


@functools.partial(
    pl.kernel,
    out_type=jax.ShapeDtypeStruct((B * C, D), jnp.float32),
    mesh=plsc.VectorSubcoreMesh(core_axis_name="c", subcore_axis_name="s"),
    compiler_params=pltpu.CompilerParams(
        needs_layout_passes=False, use_tc_tiling_on_sc=False),
    scratch_types=[
        pltpu.VMEM((RW * C,), jnp.int32),
        pltpu.VMEM((C, RW), jnp.int32),
        pltpu.VMEM((NB * C, D), jnp.float32),
        pltpu.VMEM((NB * C, D), jnp.float32),
        pltpu.VMEM((NB * C, D), jnp.float32),
        pltpu.VMEM((EV, D), jnp.float32),
        pltpu.VMEM((2, EV), jnp.float32),
        pltpu.VMEM((2, VPAD), jnp.float32),
        pltpu.VMEM((RW,), jnp.float32),
        pltpu.VMEM_SHARED((2, VPAD), jnp.float32),
        pltpu.SemaphoreType.DMA,
        pltpu.SemaphoreType.DMA,
        pltpu.SemaphoreType.DMA,
        pltpu.SemaphoreType.DMA,
        pltpu.SemaphoreType.DMA,
        pltpu.SemaphoreType.DMA,
    ],
)
def _sc_kernel(table_hbm, idx_hbm, idxt_hbm, out_hbm,
               idx_v, idxt_v, buf_a, buf_b, buf_c, tstat_v, ms_loc, ms_v,
               lse_v, ms_sh, sem_ga, sem_gb, sem_gc, sem_oa, sem_ob, sem_oc):
    _sc_body(table_hbm, idx_hbm, idxt_hbm, out_hbm,
             idx_v, idxt_v, [buf_a, buf_b, buf_c], tstat_v, ms_loc, ms_v,
             lse_v, ms_sh, [sem_ga, sem_gb, sem_gc], [sem_oa, sem_ob, sem_oc])


def kernel(batch_of_context_vectors, embedding_weight):
    idx_flat = batch_of_context_vectors.reshape(-1)
    idx_t = batch_of_context_vectors.T.reshape(-1)
    table_pad = jnp.concatenate(
        [embedding_weight, jnp.zeros((VPAD - VOCAB, D), jnp.float32)])
    out = _sc_kernel(table_pad, idx_flat, idx_t)
    return out.reshape(B, C * D)
